# baseline (speedup 1.0000x reference)
import functools

import jax
import jax.numpy as jnp
from jax import lax
from jax.experimental import pallas as pl
from jax.experimental.pallas import tpu as pltpu
from jax.experimental.pallas import tpu_sc as plsc

F = 26
V = 100000
D = 16
B = 16384

NC = 2
NS = 16
NW = NC * NS

ROWS = B * F
RPW = ROWS // NW
CHUNK = 128
NCH = RPW // CHUNK
PAT_ROWS = 13


def _esmm_kernel(idx_hbm, pat_hbm, tab_hbm, out_hbm, idx_v, pat_v, rows_v, sem):
    wid = lax.axis_index("s") * NC + lax.axis_index("c")
    base = wid * NCH
    pltpu.sync_copy(idx_hbm.at[pl.ds(base, NCH)], idx_v)
    pltpu.sync_copy(pat_hbm, pat_v)

    def body(j, carry):
        r = j % PAT_ROWS
        for v in range(CHUNK // 16):
            sl = pl.ds(v * 16, 16)
            idx_v[j, sl] = idx_v[j, sl] + pat_v[r, sl]
        pltpu.async_copy(tab_hbm.at[idx_v.at[j]], rows_v, sem).wait()
        pltpu.sync_copy(rows_v, out_hbm.at[pl.ds((base + j) * CHUNK, CHUNK)])
        return carry

    lax.fori_loop(0, NCH, body, 0)


@jax.jit
def _esmm(batch, tables):
    idx2d = batch.astype(jnp.int32).reshape(ROWS // CHUNK, CHUNK)
    tab_flat = tables.reshape(F * V, D)
    pat = ((jnp.arange(PAT_ROWS * CHUNK, dtype=jnp.int32) % F) * V).reshape(
        PAT_ROWS, CHUNK)
    mesh = plsc.VectorSubcoreMesh(core_axis_name="c", subcore_axis_name="s")
    out = pl.kernel(
        _esmm_kernel,
        out_type=jax.ShapeDtypeStruct((ROWS, D), jnp.float32),
        mesh=mesh,
        scratch_types=[
            pltpu.VMEM((NCH, CHUNK), jnp.int32),
            pltpu.VMEM((PAT_ROWS, CHUNK), jnp.int32),
            pltpu.VMEM((CHUNK, D), jnp.float32),
            pltpu.SemaphoreType.DMA,
        ],
        compiler_params=pltpu.CompilerParams(use_tc_tiling_on_sc=False),
    )(idx2d, pat, tab_flat)
    return out.reshape(B, F * D)


def kernel(batch, tables):
    return _esmm(batch, tables)


# speedup vs baseline: 1.0596x; 1.0596x over previous
import functools

import jax
import jax.numpy as jnp
from jax import lax
from jax.experimental import pallas as pl
from jax.experimental.pallas import tpu as pltpu
from jax.experimental.pallas import tpu_sc as plsc

F = 26
V = 100000
D = 16
B = 16384

NC = 2
NS = 16
NW = NC * NS

ROWS = B * F
RPW = ROWS // NW
CHUNK = 128
NCH = RPW // CHUNK
PAT_ROWS = 13


G = 13
NG = NCH // G


def _esmm_kernel(idx_hbm, pat_hbm, tab_hbm, out_hbm, idx_v, pat_v, rows_a,
                 rows_b, sem_g, sem_o):
    wid = lax.axis_index("s") * NC + lax.axis_index("c")
    base = wid * NCH
    pltpu.sync_copy(idx_hbm.at[pl.ds(base, NCH)], idx_v)
    pltpu.sync_copy(pat_hbm, pat_v)

    def add_body(j, carry):
        r = j % PAT_ROWS
        for v in range(CHUNK // 16):
            sl = pl.ds(v * 16, 16)
            idx_v[j, sl] = idx_v[j, sl] + pat_v[r, sl]
        return carry

    lax.fori_loop(0, NCH, add_body, 0)

    bufs = [rows_a, rows_b]
    out_cp = [None, None]
    for g in range(NG):
        b = g % 2
        if out_cp[b] is not None:
            out_cp[b].wait()
        cps = []
        for c in range(G):
            j = g * G + c
            cps.append(
                pltpu.async_copy(tab_hbm.at[idx_v.at[j]],
                                 bufs[b].at[pl.ds(c * CHUNK, CHUNK)], sem_g))
        for cp in cps:
            cp.wait()
        out_cp[b] = pltpu.async_copy(
            bufs[b], out_hbm.at[pl.ds((base + g * G) * CHUNK, G * CHUNK)],
            sem_o)
    out_cp[0].wait()
    out_cp[1].wait()


@jax.jit
def _esmm(batch, tables):
    idx2d = batch.astype(jnp.int32).reshape(ROWS // CHUNK, CHUNK)
    tab_flat = tables.reshape(F * V, D)
    pat = ((jnp.arange(PAT_ROWS * CHUNK, dtype=jnp.int32) % F) * V).reshape(
        PAT_ROWS, CHUNK)
    mesh = plsc.VectorSubcoreMesh(core_axis_name="c", subcore_axis_name="s")
    out = pl.kernel(
        _esmm_kernel,
        out_type=jax.ShapeDtypeStruct((ROWS, D), jnp.float32),
        mesh=mesh,
        scratch_types=[
            pltpu.VMEM((NCH, CHUNK), jnp.int32),
            pltpu.VMEM((PAT_ROWS, CHUNK), jnp.int32),
            pltpu.VMEM((G * CHUNK, D), jnp.float32),
            pltpu.VMEM((G * CHUNK, D), jnp.float32),
            pltpu.SemaphoreType.DMA,
            pltpu.SemaphoreType.DMA,
        ],
        compiler_params=pltpu.CompilerParams(use_tc_tiling_on_sc=False),
    )(idx2d, pat, tab_flat)
    return out.reshape(B, F * D)


def kernel(batch, tables):
    return _esmm(batch, tables)


# speedup vs baseline: 5.5660x; 5.2531x over previous
import functools

import jax
import jax.numpy as jnp
from jax import lax
from jax.experimental import pallas as pl
from jax.experimental.pallas import tpu as pltpu
from jax.experimental.pallas import tpu_sc as plsc

F = 26
V = 100000
D = 16
B = 16384

NC = 2
NS = 16
NW = NC * NS

P = F * D
PPW = P // NW
L = 16


HALF = B // 2


def _esmm_kernel(batch_t, tab_t, out_t, plane_v, idx_v, out_v, sem):
    wid = lax.axis_index("s") * NC + lax.axis_index("c")

    def plane_body(j, carry):
        p = wid * PPW + j
        f = p // D
        d = p % D
        pltpu.sync_copy(tab_t.at[f, d], plane_v)
        for h in range(2):
            pltpu.sync_copy(batch_t.at[f, pl.ds(h * HALF, HALF)], idx_v)

            def gather_body(i, c):
                iv = idx_v[pl.ds(i * L, L)]
                vals = plsc.load_gather(plane_v, [iv])
                out_v[pl.ds(h * HALF + i * L, L)] = vals
                return c

            lax.fori_loop(0, HALF // L, gather_body, 0)
        pltpu.async_copy(out_v, out_t.at[p], sem).wait()
        return carry

    lax.fori_loop(0, PPW, plane_body, 0)


@jax.jit
def _esmm(batch, tables):
    batch_t = batch.astype(jnp.int32).T
    tab_t = tables.transpose(0, 2, 1)
    mesh = plsc.VectorSubcoreMesh(core_axis_name="c", subcore_axis_name="s")
    out_t = pl.kernel(
        _esmm_kernel,
        out_type=jax.ShapeDtypeStruct((P, B), jnp.float32),
        mesh=mesh,
        scratch_types=[
            pltpu.VMEM((V,), jnp.float32),
            pltpu.VMEM((HALF,), jnp.int32),
            pltpu.VMEM((B,), jnp.float32),
            pltpu.SemaphoreType.DMA,
        ],
        compiler_params=pltpu.CompilerParams(
            use_tc_tiling_on_sc=True, needs_layout_passes=False),
    )(batch_t, tab_t)
    return out_t.T.reshape(B, F * D)


def kernel(batch, tables):
    return _esmm(batch, tables)
